# baseline (speedup 1.0000x reference)
import functools

import jax
import jax.numpy as jnp
from jax import lax
from jax.experimental import pallas as pl
from jax.experimental.pallas import tpu as pltpu
from jax.experimental.pallas import tpu_sc as plsc

f32 = jnp.float32
i32 = jnp.int32

NREAL = 10000
NP = 10112
D = 64
NG = 16
NW = 32
NE = 320000
STR = NP // 16
NSC = 2

KD = 80
CD = 128
EPD = NW * KD * CD

C = 64
K0 = 252
K1 = 72
G = 6
NGRP0 = K0 // (2 * G)
NGRP1 = K1 // (2 * G)
N0 = 16 * K0 * C
N1 = 16 * K1 * C

_mesh = plsc.VectorSubcoreMesh(core_axis_name="c", subcore_axis_name="s")
_sc_params = pltpu.CompilerParams(use_tc_tiling_on_sc=False)


@functools.partial(
    pl.kernel,
    out_type=jax.ShapeDtypeStruct((NSC, NP, 16), f32),
    mesh=_mesh,
    compiler_params=_sc_params,
    scratch_types=[
        pltpu.VMEM((KD, CD), i32),
        pltpu.VMEM((CD, 16), f32),
        pltpu.VMEM_SHARED((NP, 16), f32),
    ],
)
def _deg(dst_hbm, ones_hbm, zeros_hbm, out_hbm, didx, ones_v, acc):
    c = lax.axis_index("c")
    s = lax.axis_index("s")
    w = s * NSC + c
    pltpu.sync_copy(dst_hbm.at[w], didx)
    pltpu.sync_copy(ones_hbm, ones_v)
    pltpu.sync_copy(zeros_hbm.at[pl.ds(s * STR, STR)], acc.at[pl.ds(s * STR, STR)])
    plsc.subcore_barrier()

    def chunk(j, carry):
        pltpu.sync_copy(ones_v, acc.at[didx.at[j]], add=True)
        return carry

    lax.fori_loop(0, KD, chunk, 0)
    plsc.subcore_barrier()
    pltpu.sync_copy(acc.at[pl.ds(s * STR, STR)], out_hbm.at[c, pl.ds(s * STR, STR)])


@functools.partial(
    pl.kernel,
    out_type=jax.ShapeDtypeStruct((NSC, NP, D), f32),
    mesh=_mesh,
    compiler_params=_sc_params,
    scratch_types=[
        pltpu.VMEM((K0, C), i32),
        pltpu.VMEM((K0, C), i32),
        pltpu.VMEM((2 * G, C, D), f32),
        pltpu.VMEM_SHARED((NP, D), f32),
        pltpu.SemaphoreType.DMA,
        pltpu.SemaphoreType.DMA,
        pltpu.SemaphoreType.DMA,
        pltpu.SemaphoreType.DMA,
    ],
)
def _prop(y_hbm, src_hbm, dst_hbm, zeros_hbm, out_hbm, sidx, didx, rows, acc,
          gsa, gsb, ssa, ssb):
    c = lax.axis_index("c")
    s = lax.axis_index("s")
    kc = jnp.where(c == 0, K0, K1)
    ngrp = jnp.where(c == 0, NGRP0, NGRP1)
    pltpu.sync_copy(src_hbm.at[c, s], sidx)
    pltpu.sync_copy(dst_hbm.at[c, s], didx)
    pltpu.sync_copy(zeros_hbm.at[pl.ds(s * STR, STR)], acc.at[pl.ds(s * STR, STR)])
    plsc.subcore_barrier()

    for b in range(G):
        pltpu.async_copy(y_hbm.at[sidx.at[b]], rows.at[b], gsa)
        pltpu.async_copy(y_hbm.at[sidx.at[G + b]], rows.at[G + b], gsb)

    def body(i, carry):
        j = i * 2 * G

        def run_group(base, boff, gsem, ssem):
            for b in range(G):
                jb = base + b
                pltpu.make_async_copy(y_hbm.at[sidx.at[jb]], rows.at[boff + b], gsem).wait()
                pltpu.async_copy(rows.at[boff + b], acc.at[didx.at[jb]], ssem, add=True)
            for b in range(G):
                jb = base + b
                pltpu.make_async_copy(rows.at[boff + b], acc.at[didx.at[jb]], ssem).wait()

            @pl.when(base + 2 * G < kc)
            def _():
                for b in range(G):
                    pltpu.async_copy(y_hbm.at[sidx.at[base + 2 * G + b]], rows.at[boff + b], gsem)

        run_group(j, 0, gsa, ssa)
        run_group(j + G, G, gsb, ssb)
        return carry

    lax.fori_loop(0, ngrp, body, 0)
    plsc.subcore_barrier()
    pltpu.sync_copy(acc.at[pl.ds(s * STR, STR)], out_hbm.at[c, pl.ds(s * STR, STR)])


def _pre_body(x_ref, w1_ref, degp_ref, y_ref, dinv_ref):
    deg = degp_ref[0, :, 0:1] + degp_ref[1, :, 0:1] + 1.0
    dinv = lax.rsqrt(deg)
    dinv_ref[...] = dinv
    xw = jnp.dot(x_ref[...], w1_ref[...], preferred_element_type=f32)
    y_ref[...] = xw * dinv


def _mid_body(p_ref, y_ref, dinv_ref, b_ref, w2_ref, y2_ref):
    dinv = dinv_ref[...]
    t = (p_ref[0] + p_ref[1] + y_ref[...]) * dinv + b_ref[...]
    h = jnp.maximum(t, 0.0)
    y2_ref[...] = jnp.dot(h, w2_ref[...], preferred_element_type=f32) * dinv


def _post_body(p_ref, y_ref, dinv_ref, b_ref, batch_ref, fcw_ref, fcb_ref, out_ref):
    h = jnp.maximum((p_ref[0] + p_ref[1] + y_ref[...]) * dinv_ref[...] + b_ref[...], 0.0)
    gid = lax.broadcasted_iota(i32, (NG, NP), 0)
    onehot = (batch_ref[...] == gid).astype(f32)
    sums = jnp.dot(onehot, h, preferred_element_type=f32)
    counts = jnp.sum(onehot, axis=1, keepdims=True)
    pooled = sums / jnp.maximum(counts, 1.0)
    out_ref[...] = jnp.dot(pooled, fcw_ref[...], preferred_element_type=f32) + fcb_ref[...]


_pre = pl.pallas_call(
    _pre_body,
    out_shape=(
        jax.ShapeDtypeStruct((NP, D), f32),
        jax.ShapeDtypeStruct((NP, 1), f32),
    ),
)

_mid = pl.pallas_call(
    _mid_body,
    out_shape=jax.ShapeDtypeStruct((NP, D), f32),
)

_post = pl.pallas_call(
    _post_body,
    out_shape=jax.ShapeDtypeStruct((NG, 1), f32),
)


def _edge_arrays(e_flat):
    e0 = e_flat[:N0].reshape(16, K0, C)
    e1 = jnp.concatenate(
        [e_flat[N0:N0 + N1].reshape(16, K1, C),
         jnp.full((16, K0 - K1, C), NREAL, i32)], axis=1)
    return jnp.stack([e0, e1])


def kernel(x, ei, batch, W1, b1, W2, b2, fc_W, fc_b):
    ei = ei.astype(i32)
    npad = N0 + N1 - NE
    src_flat = jnp.concatenate([ei[0], jnp.full((npad,), NREAL, i32)])
    dst_flat = jnp.concatenate([ei[1], jnp.full((npad,), NREAL, i32)])
    src = _edge_arrays(src_flat)
    dst = _edge_arrays(dst_flat)
    dst_d = jnp.concatenate([ei[1], jnp.full((EPD - NE,), NREAL, i32)]).reshape(NW, KD, CD)
    x_pad = jnp.pad(x, ((0, NP - NREAL), (0, 0)))
    batch_pad = jnp.pad(batch.astype(i32), (0, NP - NREAL), constant_values=NG).reshape(1, NP)
    zeros64 = jnp.zeros((NP, D), f32)
    zeros16 = jnp.zeros((NP, 16), f32)
    ones16 = jnp.ones((CD, 16), f32)

    degp = _deg(dst_d, ones16, zeros16)
    y1, dinv = _pre(x_pad, W1, degp)
    p1 = _prop(y1, src, dst, zeros64)
    y2 = _mid(p1, y1, dinv, b1.reshape(1, D), W2)
    p2 = _prop(y2, src, dst, zeros64)
    out = _post(p2, y2, dinv, b2.reshape(1, D), batch_pad, fc_W, fc_b.reshape(1, 1))
    return out.reshape(-1)


# speedup vs baseline: 1.5233x; 1.5233x over previous
import functools

import jax
import jax.numpy as jnp
from jax import lax
from jax.experimental import pallas as pl
from jax.experimental.pallas import tpu as pltpu
from jax.experimental.pallas import tpu_sc as plsc

f32 = jnp.float32
i32 = jnp.int32

NREAL = 10000
NP = 10112
D = 64
NG = 16
NW = 32
NE = 320000
STR = NP // 16
NSC = 2

KD = 80
CD = 128
EPD = NW * KD * CD

C = 64
K = 160
G = 8
NGRP = K // (2 * G)
EP = NW * K * C

_mesh = plsc.VectorSubcoreMesh(core_axis_name="c", subcore_axis_name="s")
_sc_params = pltpu.CompilerParams(use_tc_tiling_on_sc=False)


@functools.partial(
    pl.kernel,
    out_type=jax.ShapeDtypeStruct((NSC, NP, 16), f32),
    mesh=_mesh,
    compiler_params=_sc_params,
    scratch_types=[
        pltpu.VMEM((KD, CD), i32),
        pltpu.VMEM((CD, 16), f32),
        pltpu.VMEM_SHARED((NP, 16), f32),
    ],
)
def _deg(dst_hbm, ones_hbm, zeros_hbm, out_hbm, didx, ones_v, acc):
    c = lax.axis_index("c")
    s = lax.axis_index("s")
    w = s * NSC + c
    pltpu.sync_copy(dst_hbm.at[w], didx)
    pltpu.sync_copy(ones_hbm, ones_v)
    pltpu.sync_copy(zeros_hbm.at[pl.ds(s * STR, STR)], acc.at[pl.ds(s * STR, STR)])
    plsc.subcore_barrier()

    def chunk(j, carry):
        pltpu.sync_copy(ones_v, acc.at[didx.at[j]], add=True)
        return carry

    lax.fori_loop(0, KD, chunk, 0)
    plsc.subcore_barrier()
    pltpu.sync_copy(acc.at[pl.ds(s * STR, STR)], out_hbm.at[c, pl.ds(s * STR, STR)])


@functools.partial(
    pl.kernel,
    out_type=jax.ShapeDtypeStruct((NSC, NP, D), f32),
    mesh=_mesh,
    compiler_params=_sc_params,
    scratch_types=[
        pltpu.VMEM((K, C), i32),
        pltpu.VMEM((K, C), i32),
        pltpu.VMEM((2 * G, C, D), f32),
        pltpu.VMEM_SHARED((NP, D), f32),
        pltpu.SemaphoreType.DMA,
        pltpu.SemaphoreType.DMA,
        pltpu.SemaphoreType.DMA,
        pltpu.SemaphoreType.DMA,
    ],
)
def _prop(y_hbm, src_hbm, dst_hbm, zeros_hbm, out_hbm, sidx, didx, rows, acc,
          gsa, gsb, ssa, ssb):
    c = lax.axis_index("c")
    s = lax.axis_index("s")
    w = s * NSC + c
    pltpu.sync_copy(src_hbm.at[w], sidx)
    pltpu.sync_copy(dst_hbm.at[w], didx)
    pltpu.sync_copy(zeros_hbm.at[pl.ds(s * STR, STR)], acc.at[pl.ds(s * STR, STR)])
    plsc.subcore_barrier()

    for b in range(G):
        pltpu.async_copy(y_hbm.at[sidx.at[b]], rows.at[b], gsa)
        pltpu.async_copy(y_hbm.at[sidx.at[G + b]], rows.at[G + b], gsb)

    def body(i, carry):
        j = i * 2 * G

        def run_group(base, boff, gsem, ssem):
            for b in range(G):
                jb = base + b
                pltpu.make_async_copy(y_hbm.at[sidx.at[jb]], rows.at[boff + b], gsem).wait()
                pltpu.async_copy(rows.at[boff + b], acc.at[didx.at[jb]], ssem, add=True)
            for b in range(G):
                jb = base + b
                pltpu.make_async_copy(rows.at[boff + b], acc.at[didx.at[jb]], ssem).wait()

            @pl.when(base + 2 * G < K)
            def _():
                for b in range(G):
                    pltpu.async_copy(y_hbm.at[sidx.at[base + 2 * G + b]], rows.at[boff + b], gsem)

        run_group(j, 0, gsa, ssa)
        run_group(j + G, G, gsb, ssb)
        return carry

    lax.fori_loop(0, NGRP, body, 0)
    plsc.subcore_barrier()
    pltpu.sync_copy(acc.at[pl.ds(s * STR, STR)], out_hbm.at[c, pl.ds(s * STR, STR)])


def _pre_body(x_ref, w1_ref, degp_ref, y_ref, dinv_ref):
    deg = degp_ref[0, :, 0:1] + degp_ref[1, :, 0:1] + 1.0
    dinv = lax.rsqrt(deg)
    dinv_ref[...] = dinv
    xw = jnp.dot(x_ref[...], w1_ref[...], preferred_element_type=f32)
    y_ref[...] = xw * dinv


def _mid_body(p_ref, y_ref, dinv_ref, b_ref, w2_ref, y2_ref):
    dinv = dinv_ref[...]
    t = (p_ref[0] + p_ref[1] + y_ref[...]) * dinv + b_ref[...]
    h = jnp.maximum(t, 0.0)
    y2_ref[...] = jnp.dot(h, w2_ref[...], preferred_element_type=f32) * dinv


def _post_body(p_ref, y_ref, dinv_ref, b_ref, batch_ref, fcw_ref, fcb_ref, out_ref):
    h = jnp.maximum((p_ref[0] + p_ref[1] + y_ref[...]) * dinv_ref[...] + b_ref[...], 0.0)
    gid = lax.broadcasted_iota(i32, (NG, NP), 0)
    onehot = (batch_ref[...] == gid).astype(f32)
    sums = jnp.dot(onehot, h, preferred_element_type=f32)
    counts = jnp.sum(onehot, axis=1, keepdims=True)
    pooled = sums / jnp.maximum(counts, 1.0)
    out_ref[...] = jnp.dot(pooled, fcw_ref[...], preferred_element_type=f32) + fcb_ref[...]


_pre = pl.pallas_call(
    _pre_body,
    out_shape=(
        jax.ShapeDtypeStruct((NP, D), f32),
        jax.ShapeDtypeStruct((NP, 1), f32),
    ),
)

_mid = pl.pallas_call(
    _mid_body,
    out_shape=jax.ShapeDtypeStruct((NP, D), f32),
)

_post = pl.pallas_call(
    _post_body,
    out_shape=jax.ShapeDtypeStruct((NG, 1), f32),
)


def kernel(x, ei, batch, W1, b1, W2, b2, fc_W, fc_b):
    ei = ei.astype(i32)
    pad_dst = NREAL + (jnp.arange(EP - NE, dtype=i32) % (NP - NREAL))
    src = jnp.concatenate([ei[0], jnp.full((EP - NE,), NREAL, i32)]).reshape(NW, K, C)
    dst = jnp.concatenate([ei[1], pad_dst]).reshape(NW, K, C)
    dst_d = jnp.concatenate([ei[1], pad_dst[:EPD - NE]]).reshape(NW, KD, CD)
    x_pad = jnp.pad(x, ((0, NP - NREAL), (0, 0)))
    batch_pad = jnp.pad(batch.astype(i32), (0, NP - NREAL), constant_values=NG).reshape(1, NP)
    zeros64 = jnp.zeros((NP, D), f32)
    zeros16 = jnp.zeros((NP, 16), f32)
    ones16 = jnp.ones((CD, 16), f32)

    degp = _deg(dst_d, ones16, zeros16)
    y1, dinv = _pre(x_pad, W1, degp)
    p1 = _prop(y1, src, dst, zeros64)
    y2 = _mid(p1, y1, dinv, b1.reshape(1, D), W2)
    p2 = _prop(y2, src, dst, zeros64)
    out = _post(p2, y2, dinv, b2.reshape(1, D), batch_pad, fc_W, fc_b.reshape(1, 1))
    return out.reshape(-1)
